# baseline (speedup 1.0000x reference)
import functools

import jax
import jax.numpy as jnp
from jax import lax
from jax.experimental import pallas as pl
from jax.experimental.pallas import tpu as pltpu
from jax.experimental.pallas import tpu_sc as plsc

_N_USERS = 1000000
_N_ITEMS = 100000
_N_OCC = 1000
_EMBED_DIM = 32
_B = 1024


def _sc_gather(table_flat, bias_flat, exp_idx, idx_all):
  info = plsc.get_sparse_core_info()
  nw = info.num_cores * info.num_subcores
  n = idx_all.shape[0]
  per_w = n // nw
  eper_w = per_w * _EMBED_DIM

  mesh = plsc.VectorSubcoreMesh(core_axis_name="c", subcore_axis_name="s")

  @functools.partial(
      pl.kernel,
      out_type=(
          jax.ShapeDtypeStruct((n * _EMBED_DIM,), jnp.float32),
          jax.ShapeDtypeStruct((n,), jnp.float32),
      ),
      mesh=mesh,
      compiler_params=pltpu.CompilerParams(use_tc_tiling_on_sc=False),
      scratch_types=[
          pltpu.VMEM((eper_w,), jnp.int32),
          pltpu.VMEM((per_w,), jnp.int32),
          pltpu.VMEM((eper_w,), jnp.float32),
          pltpu.VMEM((per_w,), jnp.float32),
          pltpu.SemaphoreType.DMA,
          pltpu.SemaphoreType.DMA,
      ],
  )
  def k(table_hbm, bias_hbm, eidx_hbm, idx_hbm, rows_out, bias_out,
        eidx_v, idx_v, rows_v, bias_v, sem_r, sem_b):
    wid = lax.axis_index("s") * info.num_cores + lax.axis_index("c")
    ebase = wid * eper_w
    base = wid * per_w
    pltpu.sync_copy(eidx_hbm.at[pl.ds(ebase, eper_w)], eidx_v)
    pltpu.sync_copy(idx_hbm.at[pl.ds(base, per_w)], idx_v)
    cp_r = pltpu.async_copy(table_hbm.at[eidx_v], rows_v, sem_r)
    cp_b = pltpu.async_copy(bias_hbm.at[idx_v], bias_v, sem_b)
    cp_r.wait()
    cp_b.wait()
    pltpu.sync_copy(rows_v, rows_out.at[pl.ds(ebase, eper_w)])
    pltpu.sync_copy(bias_v, bias_out.at[pl.ds(base, per_w)])

  return k(table_flat, bias_flat, exp_idx, idx_all)


def _tc_body(rows_ref, biasg_ref, out_ref):
  u = rows_ref[0 * _B:1 * _B, :]
  o = rows_ref[1 * _B:2 * _B, :]
  t = rows_ref[2 * _B:3 * _B, :]
  v = rows_ref[3 * _B:4 * _B, :]
  s = u + o + t
  cross = jnp.sum(u * o + u * t + o * t, axis=1)
  row_bias = biasg_ref[0, :] + biasg_ref[1, :] + biasg_ref[2, :]
  item_bias = biasg_ref[3, :]
  inter = lax.dot_general(
      s, v, dimension_numbers=(((1,), (1,)), ((), ())),
      preferred_element_type=jnp.float32)
  out_ref[...] = inter + (cross + row_bias)[:, None] + item_bias[None, :]


def kernel(user_code, item_code, user_occupation, item_timestamp_rank,
           feature_table, bias_table):
  u = user_code.astype(jnp.int32)
  i = item_code.astype(jnp.int32) + _N_USERS
  o = user_occupation.astype(jnp.int32) + (_N_USERS + _N_ITEMS)
  t = item_timestamp_rank.astype(jnp.int32) + (_N_USERS + _N_ITEMS + _N_OCC)
  idx_all = jnp.concatenate([u, o, t, i])
  exp_idx = (idx_all[:, None] * _EMBED_DIM
             + jnp.arange(_EMBED_DIM, dtype=jnp.int32)).reshape(-1)

  rows_flat, bias_g = _sc_gather(
      feature_table.reshape(-1), bias_table.reshape(-1), exp_idx, idx_all)

  return pl.pallas_call(
      _tc_body,
      out_shape=jax.ShapeDtypeStruct((_B, _B), jnp.float32),
  )(rows_flat.reshape(4 * _B, _EMBED_DIM), bias_g.reshape(4, _B))


# speedup vs baseline: 4.6230x; 4.6230x over previous
import functools

import jax
import jax.numpy as jnp
from jax import lax
from jax.experimental import pallas as pl
from jax.experimental.pallas import tpu as pltpu
from jax.experimental.pallas import tpu_sc as plsc

_N_USERS = 1000000
_N_ITEMS = 100000
_N_OCC = 1000
_EMBED_DIM = 32
_B = 1024
_LANE = 128


def _sc_gather(table_t, bias_flat, idx_all):
  info = plsc.get_sparse_core_info()
  nw = info.num_cores * info.num_subcores
  nl = info.num_lanes
  n = idx_all.shape[0]
  per_w = n // nw
  nchunk = per_w // nl

  mesh = plsc.VectorSubcoreMesh(core_axis_name="c", subcore_axis_name="s")

  @functools.partial(
      pl.kernel,
      out_type=(
          jax.ShapeDtypeStruct((_EMBED_DIM, n), jnp.float32),
          jax.ShapeDtypeStruct((n,), jnp.float32),
      ),
      mesh=mesh,
      compiler_params=pltpu.CompilerParams(needs_layout_passes=False),
      scratch_types=[
          pltpu.VMEM((per_w,), jnp.int32),
          pltpu.VMEM((nl, _EMBED_DIM, _LANE), jnp.float32),
          pltpu.VMEM((_EMBED_DIM, per_w), jnp.float32),
          pltpu.VMEM((per_w,), jnp.float32),
          pltpu.SemaphoreType.DMA,
          pltpu.SemaphoreType.DMA,
      ],
  )
  def k(table_hbm, bias_hbm, idx_hbm, rows_out, bias_out,
        idx_v, blk_v, rows_v, bias_v, sem_r, sem_b):
    wid = lax.axis_index("s") * info.num_cores + lax.axis_index("c")
    base = wid * per_w
    pltpu.sync_copy(idx_hbm.at[pl.ds(base, per_w)], idx_v)
    cp_b = pltpu.async_copy(bias_hbm.at[idx_v], bias_v, sem_b)

    lanes = lax.iota(jnp.int32, nl)

    def chunk_body(ch, _):
      chunk = idx_v[pl.ds(ch * nl, nl)]

      def issue(j, _):
        idx = jnp.sum(jnp.where(lanes == j, chunk, 0))
        blk = pl.multiple_of((idx // _LANE) * _LANE, _LANE)
        pltpu.async_copy(table_hbm.at[:, pl.ds(blk, _LANE)],
                         blk_v.at[j], sem_r)
        return 0

      lax.fori_loop(0, nl, issue, 0)

      def drain(j, _):
        pltpu.make_async_copy(table_hbm.at[:, pl.ds(0, _LANE)],
                              blk_v.at[0], sem_r).wait()
        return 0

      lax.fori_loop(0, nl, drain, 0)

      cvec = chunk % _LANE

      def extract(d, _):
        vals = plsc.load_gather(
            blk_v, [lanes, jnp.full((nl,), d, jnp.int32), cvec])
        rows_v[d, pl.ds(ch * nl, nl)] = vals
        return 0

      lax.fori_loop(0, _EMBED_DIM, extract, 0)
      return 0

    lax.fori_loop(0, nchunk, chunk_body, 0)
    cp_b.wait()
    pltpu.sync_copy(rows_v, rows_out.at[:, pl.ds(base, per_w)])
    pltpu.sync_copy(bias_v, bias_out.at[pl.ds(base, per_w)])

  return k(table_t, bias_flat, idx_all)


def _tc_body(rows_ref, biasg_ref, out_ref):
  u = rows_ref[:, 0 * _B:1 * _B]
  o = rows_ref[:, 1 * _B:2 * _B]
  t = rows_ref[:, 2 * _B:3 * _B]
  v = rows_ref[:, 3 * _B:4 * _B]
  s = u + o + t
  cross = jnp.sum(u * o + u * t + o * t, axis=0)
  row_bias = (biasg_ref[0 * _B:1 * _B] + biasg_ref[1 * _B:2 * _B]
              + biasg_ref[2 * _B:3 * _B])
  item_bias = biasg_ref[3 * _B:4 * _B]
  inter = lax.dot_general(
      s, v, dimension_numbers=(((0,), (0,)), ((), ())),
      preferred_element_type=jnp.float32)
  out_ref[...] = inter + (cross + row_bias)[:, None] + item_bias[None, :]


def kernel(user_code, item_code, user_occupation, item_timestamp_rank,
           feature_table, bias_table):
  u = user_code.astype(jnp.int32)
  i = item_code.astype(jnp.int32) + _N_USERS
  o = user_occupation.astype(jnp.int32) + (_N_USERS + _N_ITEMS)
  t = item_timestamp_rank.astype(jnp.int32) + (_N_USERS + _N_ITEMS + _N_OCC)
  idx_all = jnp.concatenate([u, o, t, i])

  rows_t, bias_g = _sc_gather(
      feature_table.T, bias_table.reshape(-1), idx_all)

  return pl.pallas_call(
      _tc_body,
      out_shape=jax.ShapeDtypeStruct((_B, _B), jnp.float32),
  )(rows_t, bias_g)
